# baseline (speedup 1.0000x reference)
import jax
import jax.numpy as jnp
from jax.experimental import pallas as pl
from jax.experimental.pallas import tpu as pltpu

_CH = 4


def _layer1_half(s_ref, t1b, b1_ref, w2_ref, rows):
    sb_chunks = []
    t2_chunks = []
    for r in range(_CH):
        sc = s_ref[0, r * rows:(r + 1) * rows, :].astype(jnp.bfloat16)
        sb_chunks.append(sc)
        h1 = jnp.dot(sc, t1b, preferred_element_type=jnp.float32)
        h1 = jnp.maximum(h1 + b1_ref[...], 0.0)
        t2_chunks.append(jnp.dot(h1, w2_ref[...],
                                 preferred_element_type=jnp.float32))
    return sb_chunks, t2_chunks


def _gcn_kernel(x_ref, s_ref, w1_ref, b1_ref, w2_ref, b2_ref, wp_ref,
                bp_ref, o_ref, sbf_top, t2_top):
    i = pl.program_id(1)
    half = s_ref.shape[1]
    rows = half // _CH

    t1 = jnp.dot(x_ref[0], w1_ref[...],
                 preferred_element_type=jnp.float32)
    t1b = t1.astype(jnp.bfloat16)

    @pl.when(i == 0)
    def _():
        sb_chunks, t2_chunks = _layer1_half(s_ref, t1b, b1_ref, w2_ref,
                                            rows)
        sbf_top[...] = jnp.concatenate(sb_chunks, axis=0)
        t2_top[...] = jnp.concatenate(t2_chunks, axis=0)

    @pl.when(i == 1)
    def _():
        sb_chunks, t2_chunks = _layer1_half(s_ref, t1b, b1_ref, w2_ref,
                                            rows)
        t2 = jnp.concatenate([t2_top[...]] + t2_chunks, axis=0)
        t2b = t2.astype(jnp.bfloat16)

        mx_parts = []
        sm_parts = []
        for r in range(_CH):
            st = sbf_top[r * rows:(r + 1) * rows, :]
            h2 = jnp.dot(st, t2b, preferred_element_type=jnp.float32)
            h2 = jnp.maximum(h2 + b2_ref[...], 0.0)
            mx_parts.append(jnp.max(h2, axis=0, keepdims=True))
            sm_parts.append(jnp.sum(h2, axis=0, keepdims=True))
        for sc in sb_chunks:
            h2 = jnp.dot(sc, t2b, preferred_element_type=jnp.float32)
            h2 = jnp.maximum(h2 + b2_ref[...], 0.0)
            mx_parts.append(jnp.max(h2, axis=0, keepdims=True))
            sm_parts.append(jnp.sum(h2, axis=0, keepdims=True))
        mx = jnp.max(jnp.concatenate(mx_parts, axis=0), axis=0,
                     keepdims=True)
        sm = jnp.sum(jnp.concatenate(sm_parts, axis=0), axis=0,
                     keepdims=True)

        cat = jnp.concatenate([mx, sm], axis=1)
        o_ref[0] = jnp.dot(cat, wp_ref[...],
                           preferred_element_type=jnp.float32) + bp_ref[...]


def kernel(x, support, W1, b1, W2, b2, Wp, bp):
    B, N, D_IN = x.shape
    H1 = W1.shape[1]
    H2 = W2.shape[1]
    OUT = Wp.shape[1]
    half = N // 2

    b1_2d = b1.reshape(1, H1)
    b2_2d = b2.reshape(1, H2)
    bp_2d = bp.reshape(1, OUT)

    out = pl.pallas_call(
        _gcn_kernel,
        grid=(B, 2),
        in_specs=[
            pl.BlockSpec((1, N, D_IN), lambda b, i: (b, 0, 0)),
            pl.BlockSpec((1, half, N), lambda b, i: (b, i, 0)),
            pl.BlockSpec((D_IN, H1), lambda b, i: (0, 0)),
            pl.BlockSpec((1, H1), lambda b, i: (0, 0)),
            pl.BlockSpec((H1, H2), lambda b, i: (0, 0)),
            pl.BlockSpec((1, H2), lambda b, i: (0, 0)),
            pl.BlockSpec((2 * H2, OUT), lambda b, i: (0, 0)),
            pl.BlockSpec((1, OUT), lambda b, i: (0, 0)),
        ],
        out_specs=pl.BlockSpec((1, 1, OUT), lambda b, i: (b, 0, 0)),
        out_shape=jax.ShapeDtypeStruct((B, 1, OUT), jnp.float32),
        scratch_shapes=[
            pltpu.VMEM((half, N), jnp.bfloat16),
            pltpu.VMEM((half, H2), jnp.float32),
        ],
        compiler_params=pltpu.CompilerParams(
            vmem_limit_bytes=100 * 1024 * 1024,
            dimension_semantics=("parallel", "arbitrary"),
        ),
    )(x, support, W1, b1_2d, W2, b2_2d, Wp, bp_2d)
    return out.reshape(B, OUT)


# speedup vs baseline: 1.2043x; 1.2043x over previous
import jax
import jax.numpy as jnp
from jax.experimental import pallas as pl
from jax.experimental.pallas import tpu as pltpu

_CHUNKS = 8


def _gcn_kernel(x_ref, s_ref, w1_ref, b1_ref, w2_ref, b2_ref, wp_ref,
                bp_ref, o_ref):
    n = s_ref.shape[1]
    rows = n // _CHUNKS

    t1 = jnp.dot(x_ref[0], w1_ref[...],
                 preferred_element_type=jnp.float32)
    t1b = t1.astype(jnp.bfloat16)

    sb_chunks = []
    t2_chunks = []
    for r in range(_CHUNKS):
        sc = s_ref[0, r * rows:(r + 1) * rows, :].astype(jnp.bfloat16)
        sb_chunks.append(sc)
        h1 = jnp.dot(sc, t1b, preferred_element_type=jnp.float32)
        h1 = jnp.maximum(h1 + b1_ref[...], 0.0)
        t2_chunks.append(jnp.dot(h1, w2_ref[...],
                                 preferred_element_type=jnp.float32))
    t2b = jnp.concatenate(t2_chunks, axis=0).astype(jnp.bfloat16)

    mx_parts = []
    sm_parts = []
    for r in range(_CHUNKS):
        h2 = jnp.dot(sb_chunks[r], t2b, preferred_element_type=jnp.float32)
        h2 = jnp.maximum(h2 + b2_ref[...], 0.0)
        mx_parts.append(jnp.max(h2, axis=0, keepdims=True))
        sm_parts.append(jnp.sum(h2, axis=0, keepdims=True))
    mx = jnp.max(jnp.concatenate(mx_parts, axis=0), axis=0, keepdims=True)
    sm = jnp.sum(jnp.concatenate(sm_parts, axis=0), axis=0, keepdims=True)

    cat = jnp.concatenate([mx, sm], axis=1)
    o_ref[0] = jnp.dot(cat, wp_ref[...],
                       preferred_element_type=jnp.float32) + bp_ref[...]


def kernel(x, support, W1, b1, W2, b2, Wp, bp):
    B, N, D_IN = x.shape
    H1 = W1.shape[1]
    H2 = W2.shape[1]
    OUT = Wp.shape[1]

    b1_2d = b1.reshape(1, H1)
    b2_2d = b2.reshape(1, H2)
    bp_2d = bp.reshape(1, OUT)

    out = pl.pallas_call(
        _gcn_kernel,
        grid=(B,),
        in_specs=[
            pl.BlockSpec((1, N, D_IN), lambda b: (b, 0, 0)),
            pl.BlockSpec((1, N, N), lambda b: (b, 0, 0)),
            pl.BlockSpec((D_IN, H1), lambda b: (0, 0)),
            pl.BlockSpec((1, H1), lambda b: (0, 0)),
            pl.BlockSpec((H1, H2), lambda b: (0, 0)),
            pl.BlockSpec((1, H2), lambda b: (0, 0)),
            pl.BlockSpec((2 * H2, OUT), lambda b: (0, 0)),
            pl.BlockSpec((1, OUT), lambda b: (0, 0)),
        ],
        out_specs=pl.BlockSpec((1, 1, OUT), lambda b: (b, 0, 0)),
        out_shape=jax.ShapeDtypeStruct((B, 1, OUT), jnp.float32),
        compiler_params=pltpu.CompilerParams(
            vmem_limit_bytes=100 * 1024 * 1024,
            dimension_semantics=("parallel",),
        ),
    )(x, support, W1, b1_2d, W2, b2_2d, Wp, bp_2d)
    return out.reshape(B, OUT)
